# baseline (speedup 1.0000x reference)
import functools

import jax
import jax.numpy as jnp
import numpy as np
from jax import lax
from jax.experimental import pallas as pl
from jax.experimental.pallas import tpu as pltpu
from jax.experimental.pallas import tpu_sc as plsc

LOG2_T = 19
T = 1 << LOG2_T
N_LEVELS = 16
BASE_RES = 16
PER_LEVEL_SCALE = 1.3819
N_PTS = 524288
MASK = (1 << LOG2_T) - 1
P1 = np.int32(np.uint32(2654435761).astype(np.int64) - (1 << 32))

RES = [int(np.floor(BASE_RES * PER_LEVEL_SCALE**l)) for l in range(N_LEVELS)]
PLANES = ((0, 1), (0, 2), (1, 2))

NC, NS = 2, 16
NW = NC * NS
PPW = N_PTS // NW
B = 512
NGRP = B // 16
NIB = B // 128
NCHUNK = PPW // B

F = 2
OUTD = N_LEVELS * F


def _body(x, t0, t1, t2, out, xv, idxv, wv, rowsv, outv, sem):
    tabs = (t0, t1, t2)
    wid = lax.axis_index("s") * NC + lax.axis_index("c")
    base_w = wid * PPW
    iota = lax.iota(jnp.int32, 16)

    @pl.loop(0, NCHUNK)
    def _chunk(ci):
        base = base_w + ci * B
        pltpu.sync_copy(x.at[pl.ds(base, B), :], xv)

        @pl.loop(0, N_LEVELS)
        def _level(l):
            res_f = jnp.float32(RES[0])
            for ll in range(1, N_LEVELS):
                res_f = jnp.where(l == ll, jnp.float32(RES[ll]), res_f)
            lbase = l << (LOG2_T + 1)

            @pl.loop(0, NGRP)
            def _grp_a(g):
                o16 = g * 16
                i0 = [None] * 3
                i1 = [None] * 3
                y0 = [None] * 3
                y1 = [None] * 3
                pvec = iota + o16
                for c in range(3):
                    cvec = jnp.full((16,), 1, jnp.int32) * c
                    xs = plsc.load_gather(xv, [pvec, cvec]) * res_f
                    ic = xs.astype(jnp.int32)
                    wv[pl.ds(c * B + o16, 16)] = xs - ic.astype(jnp.float32)
                    i0[c] = ic
                    i1[c] = ic + 1
                for c in (1, 2):
                    y0[c] = i0[c] * P1
                    y1[c] = y0[c] + P1
                j = 0
                for (a, b) in PLANES:
                    for (xi, ym) in ((i0[a], y0[b]), (i1[a], y0[b]),
                                     (i0[a], y1[b]), (i1[a], y1[b])):
                        e0 = (((xi ^ ym) & MASK) << 1) + lbase
                        idxv[pl.ds(2 * j * B + o16, 16)] = e0
                        idxv[pl.ds((2 * j + 1) * B + o16, 16)] = e0 + 1
                        j += 1

            handles = []
            for j in range(12):
                tp = tabs[j // 4]
                for f in range(2):
                    for k in range(NIB):
                        o = (2 * j + f) * B + k * 128
                        handles.append(pltpu.async_copy(
                            tp.at[idxv.at[pl.ds(o, 128)]],
                            rowsv.at[pl.ds(o, 128)], sem))
            for h in handles:
                h.wait()

            @pl.loop(0, NGRP)
            def _grp_b(g):
                o16 = g * 16
                pvec = iota + o16
                w = [wv[pl.ds(c * B + o16, 16)] for c in range(3)]
                prod = [None, None]
                for p, (a, b) in enumerate(PLANES):
                    wx = w[a]
                    wy = w[b]
                    for f in range(2):
                        cv = []
                        for cnr in range(4):
                            rbase = (2 * (p * 4 + cnr) + f) * B
                            cv.append(rowsv[pl.ds(rbase + o16, 16)])
                        lo = cv[0] + wx * (cv[1] - cv[0])
                        hi = cv[2] + wx * (cv[3] - cv[2])
                        fv = lo + wy * (hi - lo)
                        prod[f] = fv if p == 0 else prod[f] * fv
                col0 = l * 2
                for f in range(2):
                    colvec = jnp.full((16,), 1, jnp.int32) * (col0 + f)
                    plsc.store_scatter(outv, [pvec, colvec], prod[f])

        pltpu.sync_copy(outv, out.at[pl.ds(base, B)])


_mesh = plsc.VectorSubcoreMesh(
    core_axis_name="c", subcore_axis_name="s", num_cores=NC, num_subcores=NS)

_sc_call = functools.partial(
    pl.kernel,
    out_type=jax.ShapeDtypeStruct((N_PTS, OUTD), jnp.float32),
    mesh=_mesh,
    compiler_params=pltpu.CompilerParams(
        needs_layout_passes=False, use_tc_tiling_on_sc=False),
    scratch_types=[
        pltpu.VMEM((B, 3), jnp.float32),
        pltpu.VMEM((24 * B,), jnp.int32),
        pltpu.VMEM((3 * B,), jnp.float32),
        pltpu.VMEM((24 * B,), jnp.float32),
        pltpu.VMEM((B, OUTD), jnp.float32),
        pltpu.SemaphoreType.DMA,
    ],
)(_body)


@jax.jit
def kernel(x, table0, table1, table2):
    t0 = table0.reshape(-1)
    t1 = table1.reshape(-1)
    t2 = table2.reshape(-1)
    return _sc_call(x, t0, t1, t2)


# speedup vs baseline: 3.7190x; 3.7190x over previous
import functools

import jax
import jax.numpy as jnp
import numpy as np
from jax import lax
from jax.experimental import pallas as pl
from jax.experimental.pallas import tpu as pltpu
from jax.experimental.pallas import tpu_sc as plsc

LOG2_T = 19
T = 1 << LOG2_T
N_LEVELS = 16
BASE_RES = 16
PER_LEVEL_SCALE = 1.3819
N_PTS = 524288
MASK = (1 << LOG2_T) - 1
P1 = np.int32(np.uint32(2654435761).astype(np.int64) - (1 << 32))

RES = [int(np.floor(BASE_RES * PER_LEVEL_SCALE**l)) for l in range(N_LEVELS)]
PLANES = ((0, 1), (0, 2), (1, 2))

NC, NS = 2, 16
NW = NC * NS
PPW = N_PTS // NW
B = 512
NGRP = B // 16
NIB = B // 128
NCHUNK = PPW // B

F = 2
OUTD = N_LEVELS * F


def _body(x, t0, t1, t2, out, xv, idxv, wv, rowsv, outv, sem):
    tabs = (t0, t1, t2)
    wid = lax.axis_index("s") * NC + lax.axis_index("c")
    base_w = wid * PPW
    iota = lax.iota(jnp.int32, 16)

    @pl.loop(0, NCHUNK)
    def _chunk(ci):
        base = base_w + ci * B
        pltpu.sync_copy(x.at[pl.ds(base, B), :], xv)

        @pl.loop(0, N_LEVELS)
        def _level(l):
            res_f = jnp.float32(RES[0])
            for ll in range(1, N_LEVELS):
                res_f = jnp.where(l == ll, jnp.float32(RES[ll]), res_f)
            lbase = l << (LOG2_T + 1)

            @pl.loop(0, NGRP)
            def _grp_a(g):
                o16 = g * 16
                i0 = [None] * 3
                i1 = [None] * 3
                y0 = [None] * 3
                y1 = [None] * 3
                pvec = iota + o16
                for c in range(3):
                    cvec = jnp.full((16,), 1, jnp.int32) * c
                    xs = plsc.load_gather(xv, [pvec, cvec]) * res_f
                    ic = xs.astype(jnp.int32)
                    wv[pl.ds(c * B + o16, 16)] = xs - ic.astype(jnp.float32)
                    i0[c] = ic
                    i1[c] = ic + 1
                for c in (1, 2):
                    y0[c] = i0[c] * P1
                    y1[c] = y0[c] + P1
                j = 0
                for (a, b) in PLANES:
                    for (xi, ym) in ((i0[a], y0[b]), (i1[a], y0[b]),
                                     (i0[a], y1[b]), (i1[a], y1[b])):
                        h = (xi ^ ym) & MASK
                        e0 = lbase + ((h >> 7) << 8) + (h & 127)
                        idxv[pl.ds(2 * j * B + o16, 16)] = e0
                        idxv[pl.ds((2 * j + 1) * B + o16, 16)] = e0 + 128
                        j += 1

            handles = []
            for j in range(12):
                tp = tabs[j // 4]
                for f in range(2):
                    for k in range(NIB):
                        o = (2 * j + f) * B + k * 128
                        handles.append(pltpu.async_copy(
                            tp.at[idxv.at[pl.ds(o, 128)]],
                            rowsv.at[pl.ds(o, 128)], sem))
            for h in handles:
                h.wait()

            @pl.loop(0, NGRP)
            def _grp_b(g):
                o16 = g * 16
                pvec = iota + o16
                w = [wv[pl.ds(c * B + o16, 16)] for c in range(3)]
                prod = [None, None]
                for p, (a, b) in enumerate(PLANES):
                    wx = w[a]
                    wy = w[b]
                    for f in range(2):
                        cv = []
                        for cnr in range(4):
                            rbase = (2 * (p * 4 + cnr) + f) * B
                            cv.append(rowsv[pl.ds(rbase + o16, 16)])
                        lo = cv[0] + wx * (cv[1] - cv[0])
                        hi = cv[2] + wx * (cv[3] - cv[2])
                        fv = lo + wy * (hi - lo)
                        prod[f] = fv if p == 0 else prod[f] * fv
                col0 = l * 2
                for f in range(2):
                    colvec = jnp.full((16,), 1, jnp.int32) * (col0 + f)
                    plsc.store_scatter(outv, [pvec, colvec], prod[f])

        pltpu.sync_copy(outv, out.at[pl.ds(base, B)])


_mesh = plsc.VectorSubcoreMesh(
    core_axis_name="c", subcore_axis_name="s", num_cores=NC, num_subcores=NS)

_sc_call = functools.partial(
    pl.kernel,
    out_type=jax.ShapeDtypeStruct((N_PTS, OUTD), jnp.float32),
    mesh=_mesh,
    compiler_params=pltpu.CompilerParams(
        needs_layout_passes=False, use_tc_tiling_on_sc=False),
    scratch_types=[
        pltpu.VMEM((B, 3), jnp.float32),
        pltpu.VMEM((24 * B,), jnp.int32),
        pltpu.VMEM((3 * B,), jnp.float32),
        pltpu.VMEM((24 * B,), jnp.float32),
        pltpu.VMEM((B, OUTD), jnp.float32),
        pltpu.SemaphoreType.DMA,
    ],
)(_body)


def _to_native_flat(t):
    return jnp.transpose(
        t.reshape(N_LEVELS, T // 128, 128, F), (0, 1, 3, 2)).reshape(-1)


@jax.jit
def kernel(x, table0, table1, table2):
    return _sc_call(x, _to_native_flat(table0), _to_native_flat(table1),
                    _to_native_flat(table2))


# speedup vs baseline: 3.9009x; 1.0489x over previous
import functools

import jax
import jax.numpy as jnp
import numpy as np
from jax import lax
from jax.experimental import pallas as pl
from jax.experimental.pallas import tpu as pltpu
from jax.experimental.pallas import tpu_sc as plsc

LOG2_T = 19
T = 1 << LOG2_T
N_LEVELS = 16
BASE_RES = 16
PER_LEVEL_SCALE = 1.3819
N_PTS = 524288
MASK = (1 << LOG2_T) - 1
P1 = np.int32(np.uint32(2654435761).astype(np.int64) - (1 << 32))

RES = [int(np.floor(BASE_RES * PER_LEVEL_SCALE**l)) for l in range(N_LEVELS)]
PLANES = ((0, 1), (0, 2), (1, 2))

NC, NS = 2, 16
NW = NC * NS
PPW = N_PTS // NW
B = 512
NGRP = B // 16
NIB = B // 128
NCHUNK = PPW // B

F = 2
OUTD = N_LEVELS * F


def _body(x, t0, t1, t2, out, xv, idxv, wv, rowsv, outv, sem):
    tabs = (t0, t1, t2)
    wid = lax.axis_index("s") * NC + lax.axis_index("c")
    base_w = wid * PPW
    iota = lax.iota(jnp.int32, 16)
    SZ = 24 * B

    def phase_a(l, pb3, pb24):
        res_f = jnp.float32(RES[0])
        for ll in range(1, N_LEVELS):
            res_f = jnp.where(l == ll, jnp.float32(RES[ll]), res_f)
        lbase = l << (LOG2_T + 1)

        @pl.loop(0, NGRP)
        def _grp_a(g):
            o16 = g * 16
            i0 = [None] * 3
            i1 = [None] * 3
            y0 = [None] * 3
            y1 = [None] * 3
            pvec = iota + o16
            for c in range(3):
                cvec = jnp.full((16,), 1, jnp.int32) * c
                xs = plsc.load_gather(xv, [pvec, cvec]) * res_f
                ic = xs.astype(jnp.int32)
                wv[pl.ds(pb3 + c * B + o16, 16)] = xs - ic.astype(jnp.float32)
                i0[c] = ic
                i1[c] = ic + 1
            for c in (1, 2):
                y0[c] = i0[c] * P1
                y1[c] = y0[c] + P1
            j = 0
            for (a, b) in PLANES:
                for (xi, ym) in ((i0[a], y0[b]), (i1[a], y0[b]),
                                 (i0[a], y1[b]), (i1[a], y1[b])):
                    h = (xi ^ ym) & MASK
                    e0 = lbase + ((h >> 7) << 8) + (h & 127)
                    idxv[pl.ds(pb24 + 2 * j * B + o16, 16)] = e0
                    idxv[pl.ds(pb24 + (2 * j + 1) * B + o16, 16)] = e0 + 128
                    j += 1

    def fire(pb24):
        for j in range(12):
            tp = tabs[j // 4]
            for f in range(2):
                for k in range(NIB):
                    o = pb24 + (2 * j + f) * B + k * 128
                    pltpu.async_copy(
                        tp.at[idxv.at[pl.ds(o, 128)]],
                        rowsv.at[pl.ds(o, 128)], sem)

    def drain():
        pltpu.make_async_copy(
            t0.at[pl.ds(0, SZ)], rowsv.at[pl.ds(0, SZ)], sem).wait()

    def phase_b(l, pb3, pb24):
        col0 = l * 2

        @pl.loop(0, NGRP)
        def _grp_b(g):
            o16 = g * 16
            pvec = iota + o16
            w = [wv[pl.ds(pb3 + c * B + o16, 16)] for c in range(3)]
            prod = [None, None]
            for p, (a, b) in enumerate(PLANES):
                wx = w[a]
                wy = w[b]
                for f in range(2):
                    cv = []
                    for cnr in range(4):
                        rbase = pb24 + (2 * (p * 4 + cnr) + f) * B
                        cv.append(rowsv[pl.ds(rbase + o16, 16)])
                    lo = cv[0] + wx * (cv[1] - cv[0])
                    hi = cv[2] + wx * (cv[3] - cv[2])
                    fv = lo + wy * (hi - lo)
                    prod[f] = fv if p == 0 else prod[f] * fv
            for f in range(2):
                colvec = jnp.full((16,), 1, jnp.int32) * (col0 + f)
                plsc.store_scatter(outv, [pvec, colvec], prod[f])

    @pl.loop(0, NCHUNK)
    def _chunk(ci):
        base = base_w + ci * B
        pltpu.sync_copy(x.at[pl.ds(base, B), :], xv)
        phase_a(0, 0, 0)
        fire(0)

        @pl.loop(0, N_LEVELS)
        def _level(l):
            par = l & 1
            pb24 = par * SZ
            pb3 = par * (3 * B)
            qb24 = SZ - pb24
            qb3 = 3 * B - pb3

            @pl.when(l < N_LEVELS - 1)
            def _prefetch():
                phase_a(l + 1, qb3, qb24)

            drain()

            @pl.when(l < N_LEVELS - 1)
            def _fire_next():
                fire(qb24)

            phase_b(l, pb3, pb24)

        pltpu.sync_copy(outv, out.at[pl.ds(base, B)])


_mesh = plsc.VectorSubcoreMesh(
    core_axis_name="c", subcore_axis_name="s", num_cores=NC, num_subcores=NS)

_sc_call = functools.partial(
    pl.kernel,
    out_type=jax.ShapeDtypeStruct((N_PTS, OUTD), jnp.float32),
    mesh=_mesh,
    compiler_params=pltpu.CompilerParams(
        needs_layout_passes=False, use_tc_tiling_on_sc=False),
    scratch_types=[
        pltpu.VMEM((B, 3), jnp.float32),
        pltpu.VMEM((2 * 24 * B,), jnp.int32),
        pltpu.VMEM((2 * 3 * B,), jnp.float32),
        pltpu.VMEM((2 * 24 * B,), jnp.float32),
        pltpu.VMEM((B, OUTD), jnp.float32),
        pltpu.SemaphoreType.DMA,
    ],
)(_body)


def _to_native_flat(t):
    return jnp.transpose(
        t.reshape(N_LEVELS, T // 128, 128, F), (0, 1, 3, 2)).reshape(-1)


@jax.jit
def kernel(x, table0, table1, table2):
    return _sc_call(x, _to_native_flat(table0), _to_native_flat(table1),
                    _to_native_flat(table2))


# speedup vs baseline: 5.4097x; 1.3868x over previous
import functools

import jax
import jax.numpy as jnp
import numpy as np
from jax import lax
from jax.experimental import pallas as pl
from jax.experimental.pallas import tpu as pltpu
from jax.experimental.pallas import tpu_sc as plsc

LOG2_T = 19
T = 1 << LOG2_T
N_LEVELS = 16
BASE_RES = 16
PER_LEVEL_SCALE = 1.3819
N_PTS = 524288
MASK = (1 << LOG2_T) - 1
P1 = np.int32(np.uint32(2654435761).astype(np.int64) - (1 << 32))

RES = [int(np.floor(BASE_RES * PER_LEVEL_SCALE**l)) for l in range(N_LEVELS)]
PLANES = ((0, 1), (0, 2), (1, 2))

NC, NS = 2, 16
NW = NC * NS
PPW = N_PTS // NW
B = 512
NGRP = B // 16
NIB = B // 128
NCHUNK = PPW // B

F = 2
OUTD = N_LEVELS * F

SLV = 5
RP = [RES[l] + 1 for l in range(SLV)]
G16 = [(-(-(rp * rp) // 16)) * 16 for rp in RP]
SOFF = []
_acc = 0
for _l in range(SLV):
    _row = []
    for _p in range(3):
        _row.append(_acc)
        _acc += 2 * G16[_l]
    SOFF.append(_row)
STAGE_F32 = _acc


def _body(x, t0, t1, t2, out, xv, idxv, wv, rowsv, outv, stagev, sem):
    tabs = (t0, t1, t2)
    wid = lax.axis_index("s") * NC + lax.axis_index("c")
    base_w = wid * PPW
    iota = lax.iota(jnp.int32, 16)
    SZ = 24 * B

    for l in range(SLV):
        rp = RP[l]
        slbase = l << (LOG2_T + 1)

        @pl.loop(0, G16[l] // 16)
        def _bg(g):
            gp = g * 16 + iota
            ix = gp // rp
            iy = gp - ix * rp
            h = (ix ^ (iy * P1)) & MASK
            e0 = slbase + ((h >> 7) << 8) + (h & 127)
            idxv[pl.ds(0, 16)] = e0
            idxv[pl.ds(16, 16)] = e0 + 128
            hs = []
            for p in range(3):
                for f in range(2):
                    hs.append(pltpu.async_copy(
                        tabs[p].at[idxv.at[pl.ds(f * 16, 16)]],
                        stagev.at[pl.ds(SOFF[l][p] + f * G16[l] + g * 16, 16)],
                        sem))
            for hh in hs:
                hh.wait()

    def phase_a(l, pb3, pb24):
        res_f = jnp.float32(RES[0])
        for ll in range(1, N_LEVELS):
            res_f = jnp.where(l == ll, jnp.float32(RES[ll]), res_f)
        lbase = l << (LOG2_T + 1)

        @pl.loop(0, NGRP)
        def _grp_a(g):
            o16 = g * 16
            i0 = [None] * 3
            i1 = [None] * 3
            y0 = [None] * 3
            y1 = [None] * 3
            pvec = iota + o16
            for c in range(3):
                cvec = jnp.full((16,), 1, jnp.int32) * c
                xs = plsc.load_gather(xv, [pvec, cvec]) * res_f
                ic = xs.astype(jnp.int32)
                wv[pl.ds(pb3 + c * B + o16, 16)] = xs - ic.astype(jnp.float32)
                i0[c] = ic
                i1[c] = ic + 1
            for c in (1, 2):
                y0[c] = i0[c] * P1
                y1[c] = y0[c] + P1
            j = 0
            for (a, b) in PLANES:
                for (xi, ym) in ((i0[a], y0[b]), (i1[a], y0[b]),
                                 (i0[a], y1[b]), (i1[a], y1[b])):
                    h = (xi ^ ym) & MASK
                    e0 = lbase + ((h >> 7) << 8) + (h & 127)
                    idxv[pl.ds(pb24 + 2 * j * B + o16, 16)] = e0
                    idxv[pl.ds(pb24 + (2 * j + 1) * B + o16, 16)] = e0 + 128
                    j += 1

    def fire(pb24):
        for j in range(12):
            tp = tabs[j // 4]
            for f in range(2):
                for k in range(NIB):
                    o = pb24 + (2 * j + f) * B + k * 128
                    pltpu.async_copy(
                        tp.at[idxv.at[pl.ds(o, 128)]],
                        rowsv.at[pl.ds(o, 128)], sem)

    def drain():
        pltpu.make_async_copy(
            t0.at[pl.ds(0, SZ)], rowsv.at[pl.ds(0, SZ)], sem).wait()

    def phase_b(l, pb3, pb24):
        col0 = l * 2

        @pl.loop(0, NGRP)
        def _grp_b(g):
            o16 = g * 16
            pvec = iota + o16
            w = [wv[pl.ds(pb3 + c * B + o16, 16)] for c in range(3)]
            prod = [None, None]
            for p, (a, b) in enumerate(PLANES):
                wx = w[a]
                wy = w[b]
                for f in range(2):
                    cv = []
                    for cnr in range(4):
                        rbase = pb24 + (2 * (p * 4 + cnr) + f) * B
                        cv.append(rowsv[pl.ds(rbase + o16, 16)])
                    lo = cv[0] + wx * (cv[1] - cv[0])
                    hi = cv[2] + wx * (cv[3] - cv[2])
                    fv = lo + wy * (hi - lo)
                    prod[f] = fv if p == 0 else prod[f] * fv
            for f in range(2):
                colvec = jnp.full((16,), 1, jnp.int32) * (col0 + f)
                plsc.store_scatter(outv, [pvec, colvec], prod[f])

    @pl.loop(0, NCHUNK)
    def _chunk(ci):
        base = base_w + ci * B
        pltpu.sync_copy(x.at[pl.ds(base, B), :], xv)

        for l in range(SLV):
            res_f = jnp.float32(RES[l])
            rp = RP[l]

            @pl.loop(0, NGRP)
            def _grp_s(g):
                o16 = g * 16
                pvec = iota + o16
                i0 = [None] * 3
                i1 = [None] * 3
                w = [None] * 3
                for c in range(3):
                    cvec = jnp.full((16,), 1, jnp.int32) * c
                    xs = plsc.load_gather(xv, [pvec, cvec]) * res_f
                    ic = xs.astype(jnp.int32)
                    w[c] = xs - ic.astype(jnp.float32)
                    i0[c] = ic
                    i1[c] = ic + 1
                prod = [None, None]
                for p, (a, b) in enumerate(PLANES):
                    wx = w[a]
                    wy = w[b]
                    g00 = i0[a] * rp + i0[b]
                    g10 = i1[a] * rp + i0[b]
                    for f in range(2):
                        sbase = SOFF[l][p] + f * G16[l]
                        c00 = plsc.load_gather(stagev, [g00 + sbase])
                        c10 = plsc.load_gather(stagev, [g10 + sbase])
                        c01 = plsc.load_gather(stagev, [g00 + (sbase + 1)])
                        c11 = plsc.load_gather(stagev, [g10 + (sbase + 1)])
                        lo = c00 + wx * (c10 - c00)
                        hi = c01 + wx * (c11 - c01)
                        fv = lo + wy * (hi - lo)
                        prod[f] = fv if p == 0 else prod[f] * fv
                for f in range(2):
                    colvec = jnp.full((16,), 1, jnp.int32) * (l * 2 + f)
                    plsc.store_scatter(outv, [pvec, colvec], prod[f])

        phase_a(SLV, 0, 0)
        fire(0)

        @pl.loop(SLV, N_LEVELS)
        def _level(l):
            par = (l - SLV) & 1
            pb24 = par * SZ
            pb3 = par * (3 * B)
            qb24 = SZ - pb24
            qb3 = 3 * B - pb3

            @pl.when(l < N_LEVELS - 1)
            def _prefetch():
                phase_a(l + 1, qb3, qb24)

            drain()

            @pl.when(l < N_LEVELS - 1)
            def _fire_next():
                fire(qb24)

            phase_b(l, pb3, pb24)

        pltpu.sync_copy(outv, out.at[pl.ds(base, B)])


_mesh = plsc.VectorSubcoreMesh(
    core_axis_name="c", subcore_axis_name="s", num_cores=NC, num_subcores=NS)

_sc_call = functools.partial(
    pl.kernel,
    out_type=jax.ShapeDtypeStruct((N_PTS, OUTD), jnp.float32),
    mesh=_mesh,
    compiler_params=pltpu.CompilerParams(
        needs_layout_passes=False, use_tc_tiling_on_sc=False),
    scratch_types=[
        pltpu.VMEM((B, 3), jnp.float32),
        pltpu.VMEM((2 * 24 * B,), jnp.int32),
        pltpu.VMEM((2 * 3 * B,), jnp.float32),
        pltpu.VMEM((2 * 24 * B,), jnp.float32),
        pltpu.VMEM((B, OUTD), jnp.float32),
        pltpu.VMEM((STAGE_F32,), jnp.float32),
        pltpu.SemaphoreType.DMA,
    ],
)(_body)


def _to_native_flat(t):
    return jnp.transpose(
        t.reshape(N_LEVELS, T // 128, 128, F), (0, 1, 3, 2)).reshape(-1)


@jax.jit
def kernel(x, table0, table1, table2):
    return _sc_call(x, _to_native_flat(table0), _to_native_flat(table1),
                    _to_native_flat(table2))


# speedup vs baseline: 5.6431x; 1.0432x over previous
import functools

import jax
import jax.numpy as jnp
import numpy as np
from jax import lax
from jax.experimental import pallas as pl
from jax.experimental.pallas import tpu as pltpu
from jax.experimental.pallas import tpu_sc as plsc

LOG2_T = 19
T = 1 << LOG2_T
N_LEVELS = 16
BASE_RES = 16
PER_LEVEL_SCALE = 1.3819
N_PTS = 524288
MASK = (1 << LOG2_T) - 1
P1 = np.int32(np.uint32(2654435761).astype(np.int64) - (1 << 32))

RES = [int(np.floor(BASE_RES * PER_LEVEL_SCALE**l)) for l in range(N_LEVELS)]
PLANES = ((0, 1), (0, 2), (1, 2))

NC, NS = 2, 16
NW = NC * NS
PPW = N_PTS // NW
B = 512
NGRP = B // 16
NIB = B // 128
NCHUNK = PPW // B

F = 2
OUTD = N_LEVELS * F

SLV = 5
RP = [RES[l] + 1 for l in range(SLV)]
G16 = [(-(-(rp * rp) // 16)) * 16 for rp in RP]
SOFF = []
_acc = 0
for _l in range(SLV):
    _row = []
    for _p in range(3):
        _row.append(_acc)
        _acc += 2 * G16[_l]
    SOFF.append(_row)
STAGE_F32 = _acc


def _body(x, t0, t1, t2, out, xv, idxv, wv, rowsv, outv, stagev, sem):
    tabs = (t0, t1, t2)
    wid = lax.axis_index("s") * NC + lax.axis_index("c")
    base_w = wid * PPW
    iota = lax.iota(jnp.int32, 16)
    SZ = 24 * B

    for l in range(SLV):
        rp = RP[l]
        slbase = l << (LOG2_T + 1)

        @pl.loop(0, G16[l] // 16)
        def _bg(g):
            gp = g * 16 + iota
            ix = gp // rp
            iy = gp - ix * rp
            h = (ix ^ (iy * P1)) & MASK
            e0 = slbase + ((h >> 7) << 8) + (h & 127)
            idxv[pl.ds(0, 16)] = e0
            idxv[pl.ds(16, 16)] = e0 + 128
            hs = []
            for p in range(3):
                for f in range(2):
                    hs.append(pltpu.async_copy(
                        tabs[p].at[idxv.at[pl.ds(f * 16, 16)]],
                        stagev.at[pl.ds(SOFF[l][p] + f * G16[l] + g * 16, 16)],
                        sem))
            for hh in hs:
                hh.wait()

    def phase_a(l, pb3, pb24):
        res_f = jnp.float32(RES[0])
        for ll in range(1, N_LEVELS):
            res_f = jnp.where(l == ll, jnp.float32(RES[ll]), res_f)
        lbase = l << (LOG2_T + 1)

        @pl.loop(0, NGRP)
        def _grp_a(g):
            o16 = g * 16
            i0 = [None] * 3
            i1 = [None] * 3
            y0 = [None] * 3
            y1 = [None] * 3
            pvec = iota + o16
            for c in range(3):
                cvec = jnp.full((16,), 1, jnp.int32) * c
                xs = plsc.load_gather(xv, [pvec, cvec]) * res_f
                ic = xs.astype(jnp.int32)
                wv[pl.ds(pb3 + c * B + o16, 16)] = xs - ic.astype(jnp.float32)
                i0[c] = ic
                i1[c] = ic + 1
            for c in (1, 2):
                y0[c] = i0[c] * P1
                y1[c] = y0[c] + P1
            j = 0
            for (a, b) in PLANES:
                for (xi, ym) in ((i0[a], y0[b]), (i1[a], y0[b]),
                                 (i0[a], y1[b]), (i1[a], y1[b])):
                    h = (xi ^ ym) & MASK
                    e0 = lbase + ((h >> 7) << 8) + (h & 127)
                    idxv[pl.ds(pb24 + 2 * j * B + o16, 16)] = e0
                    idxv[pl.ds(pb24 + (2 * j + 1) * B + o16, 16)] = e0 + 128
                    j += 1

    def fire(pb24):
        for j in range(12):
            tp = tabs[j // 4]
            for f in range(2):
                for k in range(NIB):
                    o = pb24 + (2 * j + f) * B + k * 128
                    pltpu.async_copy(
                        tp.at[idxv.at[pl.ds(o, 128)]],
                        rowsv.at[pl.ds(o, 128)], sem)

    def drain():
        pltpu.make_async_copy(
            t0.at[pl.ds(0, SZ)], rowsv.at[pl.ds(0, SZ)], sem).wait()

    def phase_b(l, pb3, pb24):
        col0 = l * 2

        @pl.loop(0, NGRP)
        def _grp_b(g):
            o16 = g * 16
            pvec = iota + o16
            w = [wv[pl.ds(pb3 + c * B + o16, 16)] for c in range(3)]
            prod = [None, None]
            for p, (a, b) in enumerate(PLANES):
                wx = w[a]
                wy = w[b]
                for f in range(2):
                    cv = []
                    for cnr in range(4):
                        rbase = pb24 + (2 * (p * 4 + cnr) + f) * B
                        cv.append(rowsv[pl.ds(rbase + o16, 16)])
                    lo = cv[0] + wx * (cv[1] - cv[0])
                    hi = cv[2] + wx * (cv[3] - cv[2])
                    fv = lo + wy * (hi - lo)
                    prod[f] = fv if p == 0 else prod[f] * fv
            for f in range(2):
                col = col0 + f
                ob = ((col >> 3) * (8 * B) + (g >> 3) * 1024
                      + (col & 7) * 128 + (g & 7) * 16)
                outv[pl.ds(ob, 16)] = prod[f]

    @pl.loop(0, NCHUNK)
    def _chunk(ci):
        base = base_w + ci * B
        pltpu.sync_copy(x.at[pl.ds(base, B), :], xv)

        for l in range(SLV):
            res_f = jnp.float32(RES[l])
            rp = RP[l]

            @pl.loop(0, NGRP)
            def _grp_s(g):
                o16 = g * 16
                pvec = iota + o16
                i0 = [None] * 3
                i1 = [None] * 3
                w = [None] * 3
                for c in range(3):
                    cvec = jnp.full((16,), 1, jnp.int32) * c
                    xs = plsc.load_gather(xv, [pvec, cvec]) * res_f
                    ic = xs.astype(jnp.int32)
                    w[c] = xs - ic.astype(jnp.float32)
                    i0[c] = ic
                    i1[c] = ic + 1
                prod = [None, None]
                for p, (a, b) in enumerate(PLANES):
                    wx = w[a]
                    wy = w[b]
                    g00 = i0[a] * rp + i0[b]
                    g10 = i1[a] * rp + i0[b]
                    for f in range(2):
                        sbase = SOFF[l][p] + f * G16[l]
                        c00 = plsc.load_gather(stagev, [g00 + sbase])
                        c10 = plsc.load_gather(stagev, [g10 + sbase])
                        c01 = plsc.load_gather(stagev, [g00 + (sbase + 1)])
                        c11 = plsc.load_gather(stagev, [g10 + (sbase + 1)])
                        lo = c00 + wx * (c10 - c00)
                        hi = c01 + wx * (c11 - c01)
                        fv = lo + wy * (hi - lo)
                        prod[f] = fv if p == 0 else prod[f] * fv
                for f in range(2):
                    col = l * 2 + f
                    ob = ((col >> 3) * (8 * B) + (g >> 3) * 1024
                          + (col & 7) * 128 + (g & 7) * 16)
                    outv[pl.ds(ob, 16)] = prod[f]

        phase_a(SLV, 0, 0)
        fire(0)

        @pl.loop(SLV, N_LEVELS)
        def _level(l):
            par = (l - SLV) & 1
            pb24 = par * SZ
            pb3 = par * (3 * B)
            qb24 = SZ - pb24
            qb3 = 3 * B - pb3

            @pl.when(l < N_LEVELS - 1)
            def _prefetch():
                phase_a(l + 1, qb3, qb24)

            drain()

            @pl.when(l < N_LEVELS - 1)
            def _fire_next():
                fire(qb24)

            phase_b(l, pb3, pb24)

        nb0 = base >> 7
        for cb in range(OUTD // 8):
            pltpu.sync_copy(
                outv.at[pl.ds(cb * (8 * B), 8 * B)],
                out.at[pl.ds(cb * (8 * (N_PTS // 128) * 128) + nb0 * 1024,
                             8 * B)])


_mesh = plsc.VectorSubcoreMesh(
    core_axis_name="c", subcore_axis_name="s", num_cores=NC, num_subcores=NS)

_sc_call = functools.partial(
    pl.kernel,
    out_type=jax.ShapeDtypeStruct((N_PTS * OUTD,), jnp.float32),
    mesh=_mesh,
    compiler_params=pltpu.CompilerParams(
        needs_layout_passes=False, use_tc_tiling_on_sc=False),
    scratch_types=[
        pltpu.VMEM((B, 3), jnp.float32),
        pltpu.VMEM((2 * 24 * B,), jnp.int32),
        pltpu.VMEM((2 * 3 * B,), jnp.float32),
        pltpu.VMEM((2 * 24 * B,), jnp.float32),
        pltpu.VMEM((B * OUTD,), jnp.float32),
        pltpu.VMEM((STAGE_F32,), jnp.float32),
        pltpu.SemaphoreType.DMA,
    ],
)(_body)


def _to_native_flat(t):
    return jnp.transpose(
        t.reshape(N_LEVELS, T // 128, 128, F), (0, 1, 3, 2)).reshape(-1)


@jax.jit
def kernel(x, table0, table1, table2):
    flat = _sc_call(x, _to_native_flat(table0), _to_native_flat(table1),
                    _to_native_flat(table2))
    return flat.reshape(OUTD // 8, N_PTS // 128, 8, 128).transpose(
        1, 3, 0, 2).reshape(N_PTS, OUTD)


# speedup vs baseline: 5.7545x; 1.0197x over previous
import functools

import jax
import jax.numpy as jnp
import numpy as np
from jax import lax
from jax.experimental import pallas as pl
from jax.experimental.pallas import tpu as pltpu
from jax.experimental.pallas import tpu_sc as plsc

LOG2_T = 19
T = 1 << LOG2_T
N_LEVELS = 16
BASE_RES = 16
PER_LEVEL_SCALE = 1.3819
N_PTS = 524288
MASK = (1 << LOG2_T) - 1
P1 = np.int32(np.uint32(2654435761).astype(np.int64) - (1 << 32))

RES = [int(np.floor(BASE_RES * PER_LEVEL_SCALE**l)) for l in range(N_LEVELS)]
PLANES = ((0, 1), (0, 2), (1, 2))

NC, NS = 2, 16
NW = NC * NS
PPW = N_PTS // NW
B = 512
NGRP = B // 16
NIB = B // 128
NCHUNK = PPW // B

F = 2
OUTD = N_LEVELS * F

SLV = 5
RP = [RES[l] + 1 for l in range(SLV)]
G16 = [(-(-(rp * rp) // 16)) * 16 for rp in RP]
SOFF = []
_acc = 0
for _l in range(SLV):
    _row = []
    for _p in range(3):
        _row.append(_acc)
        _acc += 2 * G16[_l]
    SOFF.append(_row)
STAGE_F32 = _acc


def _body(x, t0, t1, t2, out, xv, idxv, wv, rowsv, outv, stagev, sem):
    tabs = (t0, t1, t2)
    wid = lax.axis_index("s") * NC + lax.axis_index("c")
    base_w = wid * PPW
    iota = lax.iota(jnp.int32, 16)
    SZ = 24 * B

    for l in range(SLV):
        rp = RP[l]
        slbase = l << (LOG2_T + 1)

        @pl.loop(0, G16[l] // 16)
        def _bg(g):
            gp = g * 16 + iota
            ix = gp // rp
            iy = gp - ix * rp
            h = (ix ^ (iy * P1)) & MASK
            e0 = slbase + ((h >> 7) << 8) + (h & 127)
            idxv[pl.ds(0, 16)] = e0
            idxv[pl.ds(16, 16)] = e0 + 128
            hs = []
            for p in range(3):
                for f in range(2):
                    hs.append(pltpu.async_copy(
                        tabs[p].at[idxv.at[pl.ds(f * 16, 16)]],
                        stagev.at[pl.ds(SOFF[l][p] + f * G16[l] + g * 16, 16)],
                        sem))
            for hh in hs:
                hh.wait()

    def phase_a(l, pb3, pb24):
        res_f = jnp.float32(RES[0])
        for ll in range(1, N_LEVELS):
            res_f = jnp.where(l == ll, jnp.float32(RES[ll]), res_f)
        lbase = l << (LOG2_T + 1)

        @pl.loop(0, NGRP)
        def _grp_a(g):
            o16 = g * 16
            i0 = [None] * 3
            i1 = [None] * 3
            y0 = [None] * 3
            y1 = [None] * 3
            pvec = iota + o16
            for c in range(3):
                cvec = jnp.full((16,), 1, jnp.int32) * c
                xs = plsc.load_gather(xv, [pvec, cvec]) * res_f
                ic = xs.astype(jnp.int32)
                wv[pl.ds(pb3 + c * B + o16, 16)] = xs - ic.astype(jnp.float32)
                i0[c] = ic
                i1[c] = ic + 1
            for c in (1, 2):
                y0[c] = i0[c] * P1
                y1[c] = y0[c] + P1
            j = 0
            for (a, b) in PLANES:
                for (xi, ym) in ((i0[a], y0[b]), (i1[a], y0[b]),
                                 (i0[a], y1[b]), (i1[a], y1[b])):
                    h = (xi ^ ym) & MASK
                    e0 = lbase + ((h >> 7) << 8) + (h & 127)
                    idxv[pl.ds(pb24 + 2 * j * B + o16, 16)] = e0
                    idxv[pl.ds(pb24 + (2 * j + 1) * B + o16, 16)] = e0 + 128
                    j += 1

    def fire(pb24):
        for j in range(12):
            tp = tabs[j // 4]
            for f in range(2):
                for k in range(NIB):
                    o = pb24 + (2 * j + f) * B + k * 128
                    pltpu.async_copy(
                        tp.at[idxv.at[pl.ds(o, 128)]],
                        rowsv.at[pl.ds(o, 128)], sem)

    def drain():
        pltpu.make_async_copy(
            t0.at[pl.ds(0, SZ)], rowsv.at[pl.ds(0, SZ)], sem).wait()

    def phase_b(l, pb3, pb24):
        col0 = l * 2

        @pl.loop(0, NGRP)
        def _grp_b(g):
            o16 = g * 16
            pvec = iota + o16
            w = [wv[pl.ds(pb3 + c * B + o16, 16)] for c in range(3)]
            prod = [None, None]
            for p, (a, b) in enumerate(PLANES):
                wx = w[a]
                wy = w[b]
                for f in range(2):
                    cv = []
                    for cnr in range(4):
                        rbase = pb24 + (2 * (p * 4 + cnr) + f) * B
                        cv.append(rowsv[pl.ds(rbase + o16, 16)])
                    lo = cv[0] + wx * (cv[1] - cv[0])
                    hi = cv[2] + wx * (cv[3] - cv[2])
                    fv = lo + wy * (hi - lo)
                    prod[f] = fv if p == 0 else prod[f] * fv
            for f in range(2):
                col = col0 + f
                ob = ((col >> 3) * (8 * B) + (g >> 3) * 1024
                      + (col & 7) * 128 + (g & 7) * 16)
                outv[pl.ds(ob, 16)] = prod[f]

    @pl.loop(0, NCHUNK)
    def _chunk(ci):
        base = base_w + ci * B
        pltpu.sync_copy(x.at[pl.ds(base, B), :], xv)

        phase_a(SLV, 0, 0)
        fire(0)

        for l in range(SLV):
            res_f = jnp.float32(RES[l])
            rp = RP[l]

            @pl.loop(0, NGRP)
            def _grp_s(g):
                o16 = g * 16
                pvec = iota + o16
                i0 = [None] * 3
                i1 = [None] * 3
                w = [None] * 3
                for c in range(3):
                    cvec = jnp.full((16,), 1, jnp.int32) * c
                    xs = plsc.load_gather(xv, [pvec, cvec]) * res_f
                    ic = xs.astype(jnp.int32)
                    w[c] = xs - ic.astype(jnp.float32)
                    i0[c] = ic
                    i1[c] = ic + 1
                prod = [None, None]
                for p, (a, b) in enumerate(PLANES):
                    wx = w[a]
                    wy = w[b]
                    g00 = i0[a] * rp + i0[b]
                    g10 = i1[a] * rp + i0[b]
                    for f in range(2):
                        sbase = SOFF[l][p] + f * G16[l]
                        c00 = plsc.load_gather(stagev, [g00 + sbase])
                        c10 = plsc.load_gather(stagev, [g10 + sbase])
                        c01 = plsc.load_gather(stagev, [g00 + (sbase + 1)])
                        c11 = plsc.load_gather(stagev, [g10 + (sbase + 1)])
                        lo = c00 + wx * (c10 - c00)
                        hi = c01 + wx * (c11 - c01)
                        fv = lo + wy * (hi - lo)
                        prod[f] = fv if p == 0 else prod[f] * fv
                for f in range(2):
                    col = l * 2 + f
                    ob = ((col >> 3) * (8 * B) + (g >> 3) * 1024
                          + (col & 7) * 128 + (g & 7) * 16)
                    outv[pl.ds(ob, 16)] = prod[f]

        @pl.loop(SLV, N_LEVELS)
        def _level(l):
            par = (l - SLV) & 1
            pb24 = par * SZ
            pb3 = par * (3 * B)
            qb24 = SZ - pb24
            qb3 = 3 * B - pb3

            @pl.when(l < N_LEVELS - 1)
            def _prefetch():
                phase_a(l + 1, qb3, qb24)

            drain()

            @pl.when(l < N_LEVELS - 1)
            def _fire_next():
                fire(qb24)

            phase_b(l, pb3, pb24)

        nb0 = base >> 7
        for cb in range(OUTD // 8):
            pltpu.sync_copy(
                outv.at[pl.ds(cb * (8 * B), 8 * B)],
                out.at[pl.ds(cb * (8 * (N_PTS // 128) * 128) + nb0 * 1024,
                             8 * B)])


_mesh = plsc.VectorSubcoreMesh(
    core_axis_name="c", subcore_axis_name="s", num_cores=NC, num_subcores=NS)

_sc_call = functools.partial(
    pl.kernel,
    out_type=jax.ShapeDtypeStruct((N_PTS * OUTD,), jnp.float32),
    mesh=_mesh,
    compiler_params=pltpu.CompilerParams(
        needs_layout_passes=False, use_tc_tiling_on_sc=False),
    scratch_types=[
        pltpu.VMEM((B, 3), jnp.float32),
        pltpu.VMEM((2 * 24 * B,), jnp.int32),
        pltpu.VMEM((2 * 3 * B,), jnp.float32),
        pltpu.VMEM((2 * 24 * B,), jnp.float32),
        pltpu.VMEM((B * OUTD,), jnp.float32),
        pltpu.VMEM((STAGE_F32,), jnp.float32),
        pltpu.SemaphoreType.DMA,
    ],
)(_body)


def _to_native_flat(t):
    return jnp.transpose(
        t.reshape(N_LEVELS, T // 128, 128, F), (0, 1, 3, 2)).reshape(-1)


@jax.jit
def kernel(x, table0, table1, table2):
    flat = _sc_call(x, _to_native_flat(table0), _to_native_flat(table1),
                    _to_native_flat(table2))
    return flat.reshape(OUTD // 8, N_PTS // 128, 8, 128).transpose(
        1, 3, 0, 2).reshape(N_PTS, OUTD)
